# baseline (speedup 1.0000x reference)
import jax
import jax.numpy as jnp
from jax import lax
from jax.experimental import pallas as pl
from jax.experimental.pallas import tpu as pltpu
from jax.experimental.pallas import tpu_sc as plsc

VOCAB = 1000000
D = 64
S = 200
B = 4096

NC, NS = 2, 16
NW = NC * NS
ROWS = B * S
RPW = ROWS // NW
GATHER = 128
STEPS = RPW // GATHER
CHUNK = 256
NCHUNK = RPW // CHUNK
GPC = CHUNK // GATHER


def _body(x_hbm, tok_hbm, pos_hbm, out_hbm, idx_v, pos_v, tok_v, sem):
    wid = lax.axis_index("s") * NC + lax.axis_index("c")
    base = wid * RPW
    pltpu.sync_copy(x_hbm.at[wid], idx_v)
    pltpu.sync_copy(pos_hbm, pos_v)

    @pl.loop(0, NCHUNK)
    def _chunk(c):
        g0 = c * GPC
        h0 = pltpu.async_copy(tok_hbm.at[idx_v.at[g0]],
                              tok_v.at[pl.ds(0, GATHER)], sem)
        h1 = pltpu.async_copy(tok_hbm.at[idx_v.at[g0 + 1]],
                              tok_v.at[pl.ds(GATHER, GATHER)], sem)
        h0.wait()
        h1.wait()

        s0 = lax.rem(c * CHUNK, S)

        @pl.loop(0, CHUNK, init_carry=s0)
        def _row(r, s):
            for cc in range(4):
                pv = pos_v[s, pl.ds(cc * 16, 16)]
                plsc.addupdate(tok_v.at[r, pl.ds(cc * 16, 16)], pv)
            s1 = s + 1
            return jnp.where(s1 == S, 0, s1)

        pltpu.sync_copy(tok_v, out_hbm.at[pl.ds(base + c * CHUNK, CHUNK)])


@jax.jit
def _run(x_r, token_emb, pos_emb):
    mesh = plsc.VectorSubcoreMesh(core_axis_name="c", subcore_axis_name="s",
                                  num_cores=NC, num_subcores=NS)
    return pl.kernel(
        _body,
        out_type=jax.ShapeDtypeStruct((ROWS, D), jnp.float32),
        mesh=mesh,
        compiler_params=pltpu.CompilerParams(use_tc_tiling_on_sc=False),
        scratch_types=[
            pltpu.VMEM((STEPS, GATHER), jnp.int32),
            pltpu.VMEM((S, D), jnp.float32),
            pltpu.VMEM((CHUNK, D), jnp.float32),
            pltpu.SemaphoreType.DMA,
        ],
    )(x_r, token_emb, pos_emb)


def kernel(x, token_emb, pos_emb):
    x_r = x.reshape(NW, STEPS, GATHER)
    out = _run(x_r, token_emb, pos_emb)
    return out.reshape(B, S, D)


# speedup vs baseline: 1.3436x; 1.3436x over previous
import jax
import jax.numpy as jnp
from jax import lax
from jax.experimental import pallas as pl
from jax.experimental.pallas import tpu as pltpu
from jax.experimental.pallas import tpu_sc as plsc

VOCAB = 1000000
D = 64
S = 200
B = 4096

NC, NS = 2, 16
NW = NC * NS
ROWS = B * S
RPW = ROWS // NW
CHUNK = S
NCHUNK = RPW // CHUNK
GATHER = 100
GPC = CHUNK // GATHER
NBUF = 4


def _body(x_hbm, tok_hbm, pos_hbm, out_hbm,
          idx_v, pos_v, b0, b1, b2, b3,
          g_sem0, g_sem1, g_sem2, g_sem3,
          s_sem0, s_sem1, s_sem2, s_sem3):
    bufs = (b0, b1, b2, b3)
    g_sems = (g_sem0, g_sem1, g_sem2, g_sem3)
    s_sems = (s_sem0, s_sem1, s_sem2, s_sem3)

    wid = lax.axis_index("s") * NC + lax.axis_index("c")
    base = wid * RPW
    pltpu.sync_copy(x_hbm.at[wid], idx_v)
    pltpu.sync_copy(pos_hbm, pos_v)

    def fire_gather(c, p):
        for j in range(GPC):
            pltpu.async_copy(tok_hbm.at[idx_v.at[c * GPC + j]],
                             bufs[p].at[pl.ds(j * GATHER, GATHER)],
                             g_sems[p])

    def wait_gather(c, p):
        for j in range(GPC):
            pltpu.make_async_copy(tok_hbm.at[idx_v.at[c * GPC + j]],
                                  bufs[p].at[pl.ds(j * GATHER, GATHER)],
                                  g_sems[p]).wait()

    def out_slice(c):
        return out_hbm.at[pl.ds(base + c * CHUNK, CHUNK)]

    def fire_store(c, p):
        pltpu.async_copy(bufs[p], out_slice(c), s_sems[p])

    def wait_store(c, p):
        pltpu.make_async_copy(bufs[p], out_slice(c), s_sems[p]).wait()

    fire_gather(0, 0)
    fire_gather(1, 1)

    @pl.loop(0, NCHUNK, step=NBUF)
    def _outer(c0):
        for bq in range(NBUF):
            c = c0 + bq
            p = bq
            q = (bq + 2) % NBUF

            @pl.when(c + 2 < NCHUNK)
            def _():
                @pl.when(c >= 2)
                def _():
                    wait_store(c - 2, q)
                fire_gather(c + 2, q)

            wait_gather(c, p)

            @pl.loop(0, CHUNK, unroll=8)
            def _row(r):
                for cc in range(D // 16):
                    pv = pos_v[r, pl.ds(cc * 16, 16)]
                    plsc.addupdate(bufs[p].at[r, pl.ds(cc * 16, 16)], pv)

            fire_store(c, p)

    for k in range(NBUF):
        c = NCHUNK - NBUF + k
        wait_store(c, c % NBUF)


@jax.jit
def _run(x_r, token_emb, pos_emb):
    mesh = plsc.VectorSubcoreMesh(core_axis_name="c", subcore_axis_name="s",
                                  num_cores=NC, num_subcores=NS)
    return pl.kernel(
        _body,
        out_type=jax.ShapeDtypeStruct((ROWS, D), jnp.float32),
        mesh=mesh,
        compiler_params=pltpu.CompilerParams(use_tc_tiling_on_sc=False),
        scratch_types=(
            [pltpu.VMEM((NCHUNK * GPC, GATHER), jnp.int32),
             pltpu.VMEM((S, D), jnp.float32)]
            + [pltpu.VMEM((CHUNK, D), jnp.float32) for _ in range(NBUF)]
            + [pltpu.SemaphoreType.DMA for _ in range(2 * NBUF)]
        ),
    )(x_r, token_emb, pos_emb)


def kernel(x, token_emb, pos_emb):
    x_r = x.reshape(NW, NCHUNK * GPC, GATHER)
    out = _run(x_r, token_emb, pos_emb)
    return out.reshape(B, S, D)
